# Initial kernel scaffold; baseline (speedup 1.0000x reference)
import jax
import jax.numpy as jnp
from jax.experimental import pallas as pl


def kernel(xs, ys, proxies):
    raise NotImplementedError("write your pallas kernel here")


# speedup vs baseline: 4.9714x; 4.9714x over previous
import functools

import jax
import jax.numpy as jnp
from jax.experimental import pallas as pl

_SIGMA = 1.0


def _proxy_nca_body(xs_ref, ys_ref, prox_ref, out_ref):
    B = xs_ref.shape[0]
    C = prox_ref.shape[0]

    prox = prox_ref[:]
    n2 = jnp.sum(prox * prox, axis=1, keepdims=True)
    norm = jnp.maximum(jnp.sqrt(n2), 1e-12)
    p_n = prox / norm
    pn2 = jnp.sum(p_n * p_n, axis=1, keepdims=True)

    xs_aug = jnp.concatenate(
        [xs_ref[:], jnp.ones((B, 1), jnp.float32)], axis=1)
    p_aug = jnp.concatenate([2.0 * p_n, -pn2], axis=1)
    s = jax.lax.dot_general(
        xs_aug, p_aug, (((1,), (1,)), ((), ())),
        precision=jax.lax.Precision.HIGHEST,
        preferred_element_type=jnp.float32) * (1.0 / _SIGMA)

    m = jnp.max(s, axis=1, keepdims=True)
    lse = m + jnp.log(jnp.sum(jnp.exp(s - m), axis=1, keepdims=True))

    col = jax.lax.broadcasted_iota(jnp.int32, (B, C), 1)
    s_pos = jnp.sum(jnp.where(col == ys_ref[:], s, 0.0),
                    axis=1, keepdims=True)

    out_ref[:, :] = jnp.sum(lse - s_pos, axis=(0, 1), keepdims=True) * (1.0 / B)


@functools.partial(jax.jit, static_argnames=())
def kernel(xs, ys, proxies):
    out = pl.pallas_call(
        _proxy_nca_body,
        out_shape=jax.ShapeDtypeStruct((1, 1), jnp.float32),
    )(xs, ys.reshape(xs.shape[0], 1), proxies)
    return out[0, 0]


# speedup vs baseline: 5.9437x; 1.1956x over previous
import functools

import jax
import jax.numpy as jnp
from jax.experimental import pallas as pl

_SIGMA = 1.0


def _proxy_nca_body(xs_ref, ys_ref, prox_ref, out_ref):
    B = xs_ref.shape[0]
    C = prox_ref.shape[0]

    prox = prox_ref[:]
    n2 = jnp.sum(prox * prox, axis=1, keepdims=True)
    scale = 2.0 / jnp.maximum(jnp.sqrt(n2), 1e-12)
    p2 = prox * scale

    g2 = jax.lax.dot_general(
        xs_ref[:], p2, (((1,), (1,)), ((), ())),
        precision=jax.lax.Precision.DEFAULT,
        preferred_element_type=jnp.float32)
    pn2_row = jax.lax.dot_general(
        jnp.ones((1, p2.shape[1]), jnp.float32), p2 * p2,
        (((1,), (1,)), ((), ())),
        precision=jax.lax.Precision.DEFAULT,
        preferred_element_type=jnp.float32) * 0.25
    s = (g2 - pn2_row) * (1.0 / _SIGMA)

    m = jnp.max(s, axis=1, keepdims=True)
    lse = m + jnp.log(jnp.sum(jnp.exp(s - m), axis=1, keepdims=True))

    col = jax.lax.broadcasted_iota(jnp.int32, (B, C), 1)
    s_pos = jnp.sum(jnp.where(col == ys_ref[:], s, 0.0),
                    axis=1, keepdims=True)

    out_ref[:, :] = jnp.sum(lse - s_pos, axis=(0, 1), keepdims=True) * (1.0 / B)


@functools.partial(jax.jit, static_argnames=())
def kernel(xs, ys, proxies):
    out = pl.pallas_call(
        _proxy_nca_body,
        out_shape=jax.ShapeDtypeStruct((1, 1), jnp.float32),
    )(xs, ys.reshape(xs.shape[0], 1), proxies)
    return out[0, 0]
